# Initial kernel scaffold; baseline (speedup 1.0000x reference)
import functools

import jax
import jax.numpy as jnp
from jax import lax
from jax.experimental import pallas as pl
from jax.experimental.pallas import tpu as pltpu
from jax.experimental.pallas import tpu_sc as plsc

N = 10000
E = 320000
BLK = 200
NSUB = 16
EP = E // NSUB
EB = 80
NB = EP // EB
RPS = N // NSUB
ZR = 25


def _head_matrix(a):
    H, C = a.shape
    flat = a.reshape(-1)
    eye = jnp.repeat(jnp.eye(H, dtype=a.dtype), C, axis=0)
    return eye * flat[:, None]


def _prep0_body(x_ref, w_ref, asrcm_ref, adstm_ref, st_ref, dt_ref):
    h = jnp.dot(x_ref[...], w_ref[...], preferred_element_type=jnp.float32)
    asrc = jnp.dot(h, asrcm_ref[...], preferred_element_type=jnp.float32)
    adst = jnp.dot(h, adstm_ref[...], preferred_element_type=jnp.float32)
    z = jnp.zeros((BLK, 12), jnp.float32)
    for c in range(2):
        st_ref[c] = jnp.concatenate(
            [asrc[:, 4 * c:4 * c + 4], z, h[:, 64 * c:64 * c + 64]], axis=1)
    dt_ref[...] = jnp.concatenate([adst, jnp.zeros((BLK, 8), jnp.float32)],
                                  axis=1)


def _prep0(x, W0, Asrc0, Adst0):
    return pl.pallas_call(
        _prep0_body,
        grid=(N // BLK,),
        in_specs=[
            pl.BlockSpec((BLK, 128), lambda i: (i, 0)),
            pl.BlockSpec((128, 128), lambda i: (0, 0)),
            pl.BlockSpec((128, 8), lambda i: (0, 0)),
            pl.BlockSpec((128, 8), lambda i: (0, 0)),
        ],
        out_specs=[
            pl.BlockSpec((2, BLK, 80), lambda i: (0, i, 0)),
            pl.BlockSpec((BLK, 16), lambda i: (i, 0)),
        ],
        out_shape=[
            jax.ShapeDtypeStruct((2, N, 80), jnp.float32),
            jax.ShapeDtypeStruct((N, 16), jnp.float32),
        ],
    )(x, W0, Asrc0, Adst0)


def _mid_body(acc_ref, b0_ref, w1_ref, asrcm_ref, adstm_ref, st_ref, dt_ref):
    cols = []
    for c in range(2):
        for k in range(4):
            wsum = acc_ref[c, :, k:k + 1] + 1e-16
            cols.append(acc_ref[c, :, 16 + 16 * k:32 + 16 * k] / wsum)
    h = jnp.concatenate(cols, axis=1) + b0_ref[...]
    h = jnp.where(h > 0, h, jnp.exp(h) - 1.0)
    h1 = jnp.dot(h, w1_ref[...], preferred_element_type=jnp.float32)
    asrc = jnp.dot(h1, asrcm_ref[...], preferred_element_type=jnp.float32)
    adst = jnp.dot(h1, adstm_ref[...], preferred_element_type=jnp.float32)
    z = jnp.zeros((BLK, 12), jnp.float32)
    for c in range(2):
        st_ref[c] = jnp.concatenate(
            [asrc[:, 4 * c:4 * c + 4], z, h1[:, 160 * c:160 * c + 160]],
            axis=1)
    dt_ref[...] = jnp.concatenate([adst, jnp.zeros((BLK, 8), jnp.float32)],
                                  axis=1)


def _mid(acc0, b0, W1, Asrc1, Adst1):
    return pl.pallas_call(
        _mid_body,
        grid=(N // BLK,),
        in_specs=[
            pl.BlockSpec((2, BLK, 80), lambda i: (0, i, 0)),
            pl.BlockSpec((1, 128), lambda i: (0, 0)),
            pl.BlockSpec((128, 320), lambda i: (0, 0)),
            pl.BlockSpec((320, 8), lambda i: (0, 0)),
            pl.BlockSpec((320, 8), lambda i: (0, 0)),
        ],
        out_specs=[
            pl.BlockSpec((2, BLK, 176), lambda i: (0, i, 0)),
            pl.BlockSpec((BLK, 16), lambda i: (i, 0)),
        ],
        out_shape=[
            jax.ShapeDtypeStruct((2, N, 176), jnp.float32),
            jax.ShapeDtypeStruct((N, 16), jnp.float32),
        ],
    )(acc0, b0, W1, Asrc1, Adst1)


def _final_body(acc_ref, b1_ref, out_ref):
    tot = jnp.zeros((BLK, 40), jnp.float32)
    for c in range(2):
        for k in range(4):
            wsum = acc_ref[c, :, k:k + 1] + 1e-16
            tot = tot + acc_ref[c, :, 16 + 40 * k:56 + 40 * k] / wsum
    out_ref[...] = tot * 0.125 + b1_ref[...]


def _final(acc1, b1):
    return pl.pallas_call(
        _final_body,
        grid=(N // BLK,),
        in_specs=[
            pl.BlockSpec((2, BLK, 176), lambda i: (0, i, 0)),
            pl.BlockSpec((1, 40), lambda i: (0, 0)),
        ],
        out_specs=pl.BlockSpec((BLK, 40), lambda i: (i, 0)),
        out_shape=jax.ShapeDtypeStruct((N, 40), jnp.float32),
    )(acc1, b1)


def _make_sc_edge(R, C):
    mesh = plsc.VectorSubcoreMesh(core_axis_name="c", subcore_axis_name="s")

    @functools.partial(
        pl.kernel,
        out_type=jax.ShapeDtypeStruct((2 * N, R), jnp.float32),
        mesh=mesh,
        scratch_types=[
            pltpu.VMEM((EB,), jnp.int32),
            pltpu.VMEM((EB,), jnp.int32),
            pltpu.VMEM((EB, R), jnp.float32),
            pltpu.VMEM((EB, 16), jnp.float32),
            pltpu.VMEM((EB, R), jnp.float32),
            pltpu.VMEM((ZR, R), jnp.float32),
            pltpu.VMEM_SHARED((N, R), jnp.float32),
            pltpu.SemaphoreType.DMA,
            pltpu.SemaphoreType.DMA,
        ],
    )
    def sc_edge(src_hbm, dst_hbm, st_hbm, dt_hbm, out_hbm,
                sidx, didx, srcbuf, dstbuf, msgbuf, zbuf, acc, sem1, sem2):
        c = lax.axis_index("c")
        s = lax.axis_index("s")
        zeros16 = jnp.zeros((16,), jnp.float32)
        iota16 = lax.iota(jnp.int32, 16)

        for j in range(ZR):
            for q in range(R // 16):
                zbuf[j, pl.ds(q * 16, 16)] = zeros16

        @pl.loop(0, RPS // ZR)
        def _zero(i):
            pltpu.sync_copy(zbuf, acc.at[pl.ds(s * RPS + i * ZR, ZR)])

        for j in range(EB):
            msgbuf[j, pl.ds(0, 16)] = zeros16

        plsc.subcore_barrier()

        @pl.loop(0, NB)
        def _block(b):
            base = s * EP + b * EB
            pltpu.sync_copy(src_hbm.at[pl.ds(base, EB)], sidx)
            pltpu.sync_copy(dst_hbm.at[pl.ds(base, EB)], didx)
            coff = c * N
            for g in range(EB // 16):
                sidx[pl.ds(g * 16, 16)] = sidx[pl.ds(g * 16, 16)] + coff
            cp1 = pltpu.async_copy(st_hbm.at[sidx], srcbuf, sem1)
            cp2 = pltpu.async_copy(dt_hbm.at[didx], dstbuf, sem2)
            cp1.wait()
            cp2.wait()
            for g in range(EB // 16):
                rows = g * 16 + iota16
                ws = []
                for k in range(4):
                    kcol = jnp.full((16,), k, jnp.int32)
                    asrc = plsc.load_gather(srcbuf, [rows, kcol])
                    adst = plsc.load_gather(dstbuf, [rows, kcol + c * 4])
                    e = asrc + adst
                    e = jnp.maximum(e, 0.2 * e)
                    w = jnp.exp(e)
                    plsc.store_scatter(msgbuf, [rows, kcol], w)
                    ws.append(w)
                for p in range(16, R):
                    pcol = jnp.full((16,), p, jnp.int32)
                    hcol = plsc.load_gather(srcbuf, [rows, pcol])
                    plsc.store_scatter(msgbuf, [rows, pcol],
                                       hcol * ws[(p - 16) // C])
            pltpu.sync_copy(msgbuf, acc.at[didx], add=True)

        plsc.subcore_barrier()

        @pl.loop(0, RPS // ZR)
        def _drain(i):
            row = s * RPS + i * ZR
            pltpu.sync_copy(acc.at[pl.ds(row, ZR)], zbuf)
            pltpu.sync_copy(zbuf, out_hbm.at[pl.ds(c * N + row, ZR)])

    return sc_edge


_sc_edge_l1 = _make_sc_edge(80, 16)
_sc_edge_l2 = _make_sc_edge(176, 40)


def kernel(x, edge_index, W0, a_src0, a_dst0, b0, W1, a_src1, a_dst1, b1):
    src = edge_index[0].astype(jnp.int32)
    dst = edge_index[1].astype(jnp.int32)
    Asrc0 = _head_matrix(a_src0)
    Adst0 = _head_matrix(a_dst0)
    Asrc1 = _head_matrix(a_src1)
    Adst1 = _head_matrix(a_dst1)

    st0, dt0 = _prep0(x, W0, Asrc0, Adst0)
    acc0 = _sc_edge_l1(src, dst, st0.reshape(2 * N, 80), dt0)
    st1, dt1 = _mid(acc0.reshape(2, N, 80), b0.reshape(1, 128),
                    W1, Asrc1, Adst1)
    acc1 = _sc_edge_l2(src, dst, st1.reshape(2 * N, 176), dt1)
    return _final(acc1.reshape(2, N, 176), b1.reshape(1, 40))


# speedup vs baseline: 12.2687x; 12.2687x over previous
import dataclasses
import functools

import jax
import jax.numpy as jnp
from jax import lax
from jax.experimental import pallas as pl
from jax.experimental.pallas import tpu as pltpu
from jax.experimental.pallas import tpu_sc as plsc

N = 10000
NP = 10240
E = 320000
BLK = 200
NSUB = 16
EP = E // NSUB
EB = 80
NB = EP // EB
RPS = NP // NSUB
ZR = 32


def _head_matrix(a):
    H, C = a.shape
    flat = a.reshape(-1)
    eye = jnp.repeat(jnp.eye(H, dtype=a.dtype), C, axis=0)
    return eye * flat[:, None]


def _prep0_body(x_ref, w_ref, asrcm_ref, adstm_ref, st_ref, dt_ref):
    h = jnp.dot(x_ref[...], w_ref[...], preferred_element_type=jnp.float32)
    asrc = jnp.dot(h, asrcm_ref[...], preferred_element_type=jnp.float32)
    adst = jnp.dot(h, adstm_ref[...], preferred_element_type=jnp.float32)
    z = jnp.zeros((BLK, 12), jnp.float32)
    for c in range(2):
        st_ref[c] = jnp.concatenate(
            [asrc[:, 4 * c:4 * c + 4], z, h[:, 64 * c:64 * c + 64]], axis=1)
    dt_ref[...] = jnp.concatenate([adst, jnp.zeros((BLK, 8), jnp.float32)],
                                  axis=1)


def _prep0(x, W0, Asrc0, Adst0):
    return pl.pallas_call(
        _prep0_body,
        grid=(N // BLK,),
        in_specs=[
            pl.BlockSpec((BLK, 128), lambda i: (i, 0)),
            pl.BlockSpec((128, 128), lambda i: (0, 0)),
            pl.BlockSpec((128, 8), lambda i: (0, 0)),
            pl.BlockSpec((128, 8), lambda i: (0, 0)),
        ],
        out_specs=[
            pl.BlockSpec((2, BLK, 80), lambda i: (0, i, 0)),
            pl.BlockSpec((BLK, 16), lambda i: (i, 0)),
        ],
        out_shape=[
            jax.ShapeDtypeStruct((2, N, 80), jnp.float32),
            jax.ShapeDtypeStruct((N, 16), jnp.float32),
        ],
    )(x, W0, Asrc0, Adst0)


def _mid_body(acc_ref, b0_ref, w1_ref, asrcm_ref, adstm_ref, st_ref, dt_ref):
    cols = []
    for c in range(2):
        for k in range(4):
            wsum = acc_ref[c, :, k:k + 1] + 1e-16
            cols.append(acc_ref[c, :, 16 + 16 * k:32 + 16 * k] / wsum)
    h = jnp.concatenate(cols, axis=1) + b0_ref[...]
    h = jnp.where(h > 0, h, jnp.exp(h) - 1.0)
    h1 = jnp.dot(h, w1_ref[...], preferred_element_type=jnp.float32)
    asrc = jnp.dot(h1, asrcm_ref[...], preferred_element_type=jnp.float32)
    adst = jnp.dot(h1, adstm_ref[...], preferred_element_type=jnp.float32)
    z = jnp.zeros((BLK, 14), jnp.float32)
    for t in range(4):
        hd = 4 * (t // 2) + 2 * (t % 2)
        st_ref[t] = jnp.concatenate(
            [asrc[:, hd:hd + 2], z, h1[:, 40 * hd:40 * hd + 80]], axis=1)
    dt_ref[...] = jnp.concatenate([adst, jnp.zeros((BLK, 8), jnp.float32)],
                                  axis=1)


def _mid(acc0, b0, W1, Asrc1, Adst1):
    return pl.pallas_call(
        _mid_body,
        grid=(N // BLK,),
        in_specs=[
            pl.BlockSpec((2, BLK, 80), lambda i: (0, i, 0)),
            pl.BlockSpec((1, 128), lambda i: (0, 0)),
            pl.BlockSpec((128, 320), lambda i: (0, 0)),
            pl.BlockSpec((320, 8), lambda i: (0, 0)),
            pl.BlockSpec((320, 8), lambda i: (0, 0)),
        ],
        out_specs=[
            pl.BlockSpec((4, BLK, 96), lambda i: (0, i, 0)),
            pl.BlockSpec((BLK, 16), lambda i: (i, 0)),
        ],
        out_shape=[
            jax.ShapeDtypeStruct((4, N, 96), jnp.float32),
            jax.ShapeDtypeStruct((N, 16), jnp.float32),
        ],
    )(acc0, b0, W1, Asrc1, Adst1)


def _final_body(acc0_ref, acc1_ref, b1_ref, out_ref):
    tot = jnp.zeros((BLK, 40), jnp.float32)
    for c in range(2):
        for q, ref in ((0, acc0_ref), (1, acc1_ref)):
            for k in range(2):
                wsum = ref[c, :, k:k + 1] + 1e-16
                tot = tot + ref[c, :, 16 + 40 * k:56 + 40 * k] / wsum
    out_ref[...] = tot * 0.125 + b1_ref[...]


def _final(accq0, accq1, b1):
    return pl.pallas_call(
        _final_body,
        grid=(N // BLK,),
        in_specs=[
            pl.BlockSpec((2, BLK, 96), lambda i: (0, i, 0)),
            pl.BlockSpec((2, BLK, 96), lambda i: (0, i, 0)),
            pl.BlockSpec((1, 40), lambda i: (0, 0)),
        ],
        out_specs=pl.BlockSpec((BLK, 40), lambda i: (i, 0)),
        out_shape=jax.ShapeDtypeStruct((N, 40), jnp.float32),
    )(accq0, accq1, b1)


@functools.lru_cache(maxsize=None)
def _make_sc_edge(R, C, KH, q, tgroups):
    mesh = plsc.VectorSubcoreMesh(core_axis_name="c", subcore_axis_name="s")
    cp = pltpu.CompilerParams()
    if "needs_layout_passes" in pltpu.CompilerParams.__dataclass_fields__:
        cp = dataclasses.replace(cp, needs_layout_passes=False)
    if "use_tc_tiling_on_sc" in pltpu.CompilerParams.__dataclass_fields__:
        cp = dataclasses.replace(cp, use_tc_tiling_on_sc=False)

    @functools.partial(
        pl.kernel,
        out_type=jax.ShapeDtypeStruct((2 * NP, R), jnp.float32),
        mesh=mesh,
        compiler_params=cp,
        scratch_types=[
            pltpu.VMEM((EB,), jnp.int32),
            pltpu.VMEM((EB,), jnp.int32),
            pltpu.VMEM((EB, R), jnp.float32),
            pltpu.VMEM((EB, 16), jnp.float32),
            pltpu.VMEM((EB, R), jnp.float32),
            pltpu.VMEM((ZR, R), jnp.float32),
            pltpu.VMEM_SHARED((NP, R), jnp.float32),
            pltpu.SemaphoreType.DMA,
            pltpu.SemaphoreType.DMA,
        ],
    )
    def sc_edge(src_hbm, dst_hbm, st_hbm, dt_hbm, out_hbm,
                sidx, didx, srcbuf, dstbuf, msgbuf, zbuf, acc, sem1, sem2):
        c = lax.axis_index("c")
        hd0 = c * 4 + q * KH
        s = lax.axis_index("s")
        zeros16 = jnp.zeros((16,), jnp.float32)
        iota16 = lax.iota(jnp.int32, 16)

        for j in range(ZR):
            for col0 in range(R // 16):
                zbuf[j, pl.ds(col0 * 16, 16)] = zeros16

        @pl.loop(0, RPS // ZR)
        def _zero(i):
            pltpu.sync_copy(zbuf, acc.at[pl.ds(s * RPS + i * ZR, ZR)])

        for j in range(EB):
            msgbuf[j, pl.ds(0, 16)] = zeros16

        plsc.subcore_barrier()

        @pl.loop(0, NB)
        def _block(b):
            base = s * EP + b * EB
            pltpu.sync_copy(src_hbm.at[pl.ds(base, EB)], sidx)
            pltpu.sync_copy(dst_hbm.at[pl.ds(base, EB)], didx)
            coff = (c * (tgroups // 2) + q) * N
            for g in range(EB // 16):
                sidx[pl.ds(g * 16, 16)] = sidx[pl.ds(g * 16, 16)] + coff
            cp1 = pltpu.async_copy(st_hbm.at[sidx], srcbuf, sem1)
            cp2 = pltpu.async_copy(dt_hbm.at[didx], dstbuf, sem2)
            cp1.wait()
            cp2.wait()
            for g in range(EB // 16):
                rows = g * 16 + iota16
                ws = []
                for k in range(KH):
                    kcol = jnp.full((16,), k, jnp.int32)
                    asrc = plsc.load_gather(srcbuf, [rows, kcol])
                    adst = plsc.load_gather(dstbuf, [rows, kcol + hd0])
                    e = asrc + adst
                    e = jnp.maximum(e, 0.2 * e)
                    w = jnp.exp(e)
                    plsc.store_scatter(msgbuf, [rows, kcol], w)
                    ws.append(w)
                for p in range(16, R):
                    pcol = jnp.full((16,), p, jnp.int32)
                    hcol = plsc.load_gather(srcbuf, [rows, pcol])
                    plsc.store_scatter(msgbuf, [rows, pcol],
                                       hcol * ws[(p - 16) // C])
            pltpu.sync_copy(msgbuf, acc.at[didx], add=True)

        plsc.subcore_barrier()

        @pl.loop(0, RPS // ZR)
        def _drain(i):
            row = s * RPS + i * ZR
            pltpu.sync_copy(acc.at[pl.ds(row, ZR)], zbuf)
            pltpu.sync_copy(zbuf, out_hbm.at[pl.ds(c * NP + row, ZR)])

    return sc_edge


def kernel(x, edge_index, W0, a_src0, a_dst0, b0, W1, a_src1, a_dst1, b1):
    src = edge_index[0].astype(jnp.int32)
    dst = edge_index[1].astype(jnp.int32)
    Asrc0 = _head_matrix(a_src0)
    Adst0 = _head_matrix(a_dst0)
    Asrc1 = _head_matrix(a_src1)
    Adst1 = _head_matrix(a_dst1)

    st0, dt0 = _prep0(x, W0, Asrc0, Adst0)
    acc0 = _make_sc_edge(80, 16, 4, 0, 2)(
        src, dst, st0.reshape(2 * N, 80), dt0)
    st1, dt1 = _mid(acc0.reshape(2, NP, 80), b0.reshape(1, 128),
                    W1, Asrc1, Adst1)
    st1f = st1.reshape(4 * N, 96)
    accq0 = _make_sc_edge(96, 40, 2, 0, 4)(src, dst, st1f, dt1)
    accq1 = _make_sc_edge(96, 40, 2, 1, 4)(src, dst, st1f, dt1)
    return _final(accq0.reshape(2, NP, 96), accq1.reshape(2, NP, 96),
                  b1.reshape(1, 40))


# speedup vs baseline: 15.4116x; 1.2562x over previous
import dataclasses
import functools

import jax
import jax.numpy as jnp
from jax import lax
from jax.experimental import pallas as pl
from jax.experimental.pallas import tpu as pltpu
from jax.experimental.pallas import tpu_sc as plsc

N = 10000
NP = 10240
E = 320000
BLK = 200
NSUB = 16
EP = E // NSUB
EB = 80
NB = EP // EB
RPS = NP // NSUB
ZR = 32


def _head_matrix(a):
    H, C = a.shape
    flat = a.reshape(-1)
    eye = jnp.repeat(jnp.eye(H, dtype=a.dtype), C, axis=0)
    return eye * flat[:, None]


def _prep0_body(x_ref, w_ref, asrcm_ref, adstm_ref, st_ref, dt_ref):
    h = jnp.dot(x_ref[...], w_ref[...], preferred_element_type=jnp.float32)
    asrc = jnp.dot(h, asrcm_ref[...], preferred_element_type=jnp.float32)
    adst = jnp.dot(h, adstm_ref[...], preferred_element_type=jnp.float32)
    z = jnp.zeros((BLK, 12), jnp.float32)
    for c in range(2):
        st_ref[c] = jnp.concatenate(
            [asrc[:, 4 * c:4 * c + 4], z, h[:, 64 * c:64 * c + 64]], axis=1)
    dt_ref[...] = jnp.concatenate([adst, jnp.zeros((BLK, 8), jnp.float32)],
                                  axis=1)


def _prep0(x, W0, Asrc0, Adst0):
    return pl.pallas_call(
        _prep0_body,
        grid=(N // BLK,),
        in_specs=[
            pl.BlockSpec((BLK, 128), lambda i: (i, 0)),
            pl.BlockSpec((128, 128), lambda i: (0, 0)),
            pl.BlockSpec((128, 8), lambda i: (0, 0)),
            pl.BlockSpec((128, 8), lambda i: (0, 0)),
        ],
        out_specs=[
            pl.BlockSpec((2, BLK, 80), lambda i: (0, i, 0)),
            pl.BlockSpec((BLK, 16), lambda i: (i, 0)),
        ],
        out_shape=[
            jax.ShapeDtypeStruct((2, N, 80), jnp.float32),
            jax.ShapeDtypeStruct((N, 16), jnp.float32),
        ],
    )(x, W0, Asrc0, Adst0)


def _mid_body(acc_ref, b0_ref, w1_ref, asrcm_ref, adstm_ref, st_ref, dt_ref):
    cols = []
    for c in range(2):
        for k in range(4):
            wsum = acc_ref[c, :, k:k + 1] + 1e-16
            cols.append(acc_ref[c, :, 16 + 16 * k:32 + 16 * k] / wsum)
    h = jnp.concatenate(cols, axis=1) + b0_ref[...]
    h = jnp.where(h > 0, h, jnp.exp(h) - 1.0)
    h1 = jnp.dot(h, w1_ref[...], preferred_element_type=jnp.float32)
    asrc = jnp.dot(h1, asrcm_ref[...], preferred_element_type=jnp.float32)
    adst = jnp.dot(h1, adstm_ref[...], preferred_element_type=jnp.float32)
    z = jnp.zeros((BLK, 14), jnp.float32)
    for t in range(4):
        hd = 4 * (t // 2) + 2 * (t % 2)
        st_ref[t] = jnp.concatenate(
            [asrc[:, hd:hd + 2], z, h1[:, 40 * hd:40 * hd + 80]], axis=1)
    dt_ref[...] = jnp.concatenate([adst, jnp.zeros((BLK, 8), jnp.float32)],
                                  axis=1)


def _mid(acc0, b0, W1, Asrc1, Adst1):
    return pl.pallas_call(
        _mid_body,
        grid=(N // BLK,),
        in_specs=[
            pl.BlockSpec((2, BLK, 80), lambda i: (0, i, 0)),
            pl.BlockSpec((1, 128), lambda i: (0, 0)),
            pl.BlockSpec((128, 320), lambda i: (0, 0)),
            pl.BlockSpec((320, 8), lambda i: (0, 0)),
            pl.BlockSpec((320, 8), lambda i: (0, 0)),
        ],
        out_specs=[
            pl.BlockSpec((4, BLK, 96), lambda i: (0, i, 0)),
            pl.BlockSpec((BLK, 16), lambda i: (i, 0)),
        ],
        out_shape=[
            jax.ShapeDtypeStruct((4, N, 96), jnp.float32),
            jax.ShapeDtypeStruct((N, 16), jnp.float32),
        ],
    )(acc0, b0, W1, Asrc1, Adst1)


def _final_body(acc0_ref, acc1_ref, b1_ref, out_ref):
    tot = jnp.zeros((BLK, 40), jnp.float32)
    for c in range(2):
        for q, ref in ((0, acc0_ref), (1, acc1_ref)):
            for k in range(2):
                wsum = ref[c, :, k:k + 1] + 1e-16
                tot = tot + ref[c, :, 16 + 40 * k:56 + 40 * k] / wsum
    out_ref[...] = tot * 0.125 + b1_ref[...]


def _final(accq0, accq1, b1):
    return pl.pallas_call(
        _final_body,
        grid=(N // BLK,),
        in_specs=[
            pl.BlockSpec((2, BLK, 96), lambda i: (0, i, 0)),
            pl.BlockSpec((2, BLK, 96), lambda i: (0, i, 0)),
            pl.BlockSpec((1, 40), lambda i: (0, 0)),
        ],
        out_specs=pl.BlockSpec((BLK, 40), lambda i: (i, 0)),
        out_shape=jax.ShapeDtypeStruct((N, 40), jnp.float32),
    )(accq0, accq1, b1)


@functools.lru_cache(maxsize=None)
def _make_sc_edge(R, C, KH, q, tgroups):
    mesh = plsc.VectorSubcoreMesh(core_axis_name="c", subcore_axis_name="s")
    cp = pltpu.CompilerParams()
    if "needs_layout_passes" in pltpu.CompilerParams.__dataclass_fields__:
        cp = dataclasses.replace(cp, needs_layout_passes=False)
    if "use_tc_tiling_on_sc" in pltpu.CompilerParams.__dataclass_fields__:
        cp = dataclasses.replace(cp, use_tc_tiling_on_sc=False)

    @functools.partial(
        pl.kernel,
        out_type=jax.ShapeDtypeStruct((2 * NP, R), jnp.float32),
        mesh=mesh,
        compiler_params=cp,
        scratch_types=[
            pltpu.VMEM((2, EB), jnp.int32),
            pltpu.VMEM((2, EB), jnp.int32),
            pltpu.VMEM((2, EB), jnp.int32),
            pltpu.VMEM((2, EB, R), jnp.float32),
            pltpu.VMEM((2, EB, 16), jnp.float32),
            pltpu.VMEM((2, EB, R), jnp.float32),
            pltpu.VMEM((ZR, R), jnp.float32),
            pltpu.VMEM_SHARED((NP, R), jnp.float32),
        ] + [pltpu.SemaphoreType.DMA] * 10,
    )
    def sc_edge(src_hbm, dst_hbm, st_hbm, dt_hbm, out_hbm,
                sidx, didxg, didxs, srcbuf, dstbuf, msgbuf, zbuf, acc,
                gs0, gs1, gd0, gd1, is0, is1, id0, id1, ss0, ss1):
        c = lax.axis_index("c")
        hd0 = c * 4 + q * KH
        s = lax.axis_index("s")
        coff = (c * (tgroups // 2) + q) * N
        zeros16 = jnp.zeros((16,), jnp.float32)
        iota16 = lax.iota(jnp.int32, 16)
        gsem = (gs0, gs1)
        gdem = (gd0, gd1)
        isem = (is0, is1)
        idem = (id0, id1)
        ssem = (ss0, ss1)

        for j in range(ZR):
            for col0 in range(R // 16):
                zbuf[j, pl.ds(col0 * 16, 16)] = zeros16

        @pl.loop(0, RPS // ZR)
        def _zero(i):
            pltpu.sync_copy(zbuf, acc.at[pl.ds(s * RPS + i * ZR, ZR)])

        for par in range(2):
            for j in range(EB):
                msgbuf[par, j, pl.ds(0, 16)] = zeros16

        plsc.subcore_barrier()

        ebase = s * EP

        def adjust(par):
            for g in range(EB // 16):
                sl = pl.ds(g * 16, 16)
                sidx[par, sl] = sidx[par, sl] + coff

        def issue_gathers(par):
            pltpu.async_copy(st_hbm.at[sidx.at[par]], srcbuf.at[par],
                             gsem[par])
            pltpu.async_copy(dt_hbm.at[didxg.at[par]], dstbuf.at[par],
                             gdem[par])

        def wait_gathers(par):
            pltpu.make_async_copy(st_hbm.at[sidx.at[par]], srcbuf.at[par],
                                  gsem[par]).wait()
            pltpu.make_async_copy(dt_hbm.at[didxg.at[par]], dstbuf.at[par],
                                  gdem[par]).wait()

        def issue_idx(par, blk):
            pltpu.async_copy(src_hbm.at[pl.ds(ebase + blk * EB, EB)],
                             sidx.at[par], isem[par])
            pltpu.async_copy(dst_hbm.at[pl.ds(ebase + blk * EB, EB)],
                             didxg.at[par], idem[par])

        def wait_idx(par, blk):
            pltpu.make_async_copy(src_hbm.at[pl.ds(ebase + blk * EB, EB)],
                                  sidx.at[par], isem[par]).wait()
            pltpu.make_async_copy(dst_hbm.at[pl.ds(ebase + blk * EB, EB)],
                                  didxg.at[par], idem[par]).wait()

        def wait_scatter(par):
            pltpu.make_async_copy(msgbuf.at[par], acc.at[didxs.at[par]],
                                  ssem[par]).wait()

        def compute(par):
            for g in range(EB // 16):
                rows = g * 16 + iota16
                ws = []
                for k in range(KH):
                    kcol = jnp.full((16,), k, jnp.int32)
                    asrc = plsc.load_gather(srcbuf.at[par], [rows, kcol])
                    adst = plsc.load_gather(dstbuf.at[par],
                                            [rows, kcol + hd0])
                    e = asrc + adst
                    e = jnp.maximum(e, 0.2 * e)
                    w = jnp.exp(e)
                    plsc.store_scatter(msgbuf.at[par], [rows, kcol], w)
                    ws.append(w)
                for p in range(16, R):
                    pcol = jnp.full((16,), p, jnp.int32)
                    hcol = plsc.load_gather(srcbuf.at[par], [rows, pcol])
                    plsc.store_scatter(msgbuf.at[par], [rows, pcol],
                                       hcol * ws[(p - 16) // C])

        pltpu.sync_copy(src_hbm.at[pl.ds(ebase, EB)], sidx.at[0])
        pltpu.sync_copy(dst_hbm.at[pl.ds(ebase, EB)], didxg.at[0])
        adjust(0)
        issue_gathers(0)
        issue_idx(1, 1)

        @pl.loop(0, NB // 2)
        def _pair(i):
            for P in range(2):
                b = 2 * i + P
                Q = 1 - P
                wait_gathers(P)
                if P == 0:
                    wait_idx(Q, b + 1)
                    adjust(Q)
                    issue_gathers(Q)
                else:
                    @pl.when(i < NB // 2 - 1)
                    def _():
                        wait_idx(Q, b + 1)
                        adjust(Q)
                        issue_gathers(Q)

                @pl.when(i > 0)
                def _():
                    wait_scatter(P)
                for g in range(EB // 16):
                    sl = pl.ds(g * 16, 16)
                    didxs[P, sl] = didxg[P, sl]

                @pl.when(i < NB // 2 - 1)
                def _():
                    issue_idx(P, b + 2)

                compute(P)
                pltpu.async_copy(msgbuf.at[P], acc.at[didxs.at[P]], ssem[P],
                                 add=True)

        wait_scatter(0)
        wait_scatter(1)
        plsc.subcore_barrier()

        @pl.loop(0, RPS // ZR)
        def _drain(i):
            row = s * RPS + i * ZR
            pltpu.sync_copy(acc.at[pl.ds(row, ZR)], zbuf)
            pltpu.sync_copy(zbuf, out_hbm.at[pl.ds(c * NP + row, ZR)])

    return sc_edge


def kernel(x, edge_index, W0, a_src0, a_dst0, b0, W1, a_src1, a_dst1, b1):
    src = edge_index[0].astype(jnp.int32)
    dst = edge_index[1].astype(jnp.int32)
    Asrc0 = _head_matrix(a_src0)
    Adst0 = _head_matrix(a_dst0)
    Asrc1 = _head_matrix(a_src1)
    Adst1 = _head_matrix(a_dst1)

    st0, dt0 = _prep0(x, W0, Asrc0, Adst0)
    acc0 = _make_sc_edge(80, 16, 4, 0, 2)(
        src, dst, st0.reshape(2 * N, 80), dt0)
    st1, dt1 = _mid(acc0.reshape(2, NP, 80), b0.reshape(1, 128),
                    W1, Asrc1, Adst1)
    st1f = st1.reshape(4 * N, 96)
    accq0 = _make_sc_edge(96, 40, 2, 0, 4)(src, dst, st1f, dt1)
    accq1 = _make_sc_edge(96, 40, 2, 1, 4)(src, dst, st1f, dt1)
    return _final(accq0.reshape(2, NP, 96), accq1.reshape(2, NP, 96),
                  b1.reshape(1, 40))


# speedup vs baseline: 79.6637x; 5.1691x over previous
import dataclasses
import functools

import jax
import jax.numpy as jnp
from jax import lax
from jax.experimental import pallas as pl
from jax.experimental.pallas import tpu as pltpu
from jax.experimental.pallas import tpu_sc as plsc

N = 10000
NP = 10240
E = 320000
BLK = 200
NSUB = 16
EP = E // NSUB
EB = 80
NB = EP // EB
RPS = NP // NSUB
ZR = 32


def _head_matrix(a):
    H, C = a.shape
    flat = a.reshape(-1)
    eye = jnp.repeat(jnp.eye(H, dtype=a.dtype), C, axis=0)
    return eye * flat[:, None]


def _prep0_body(x_ref, w_ref, asrcm_ref, adstm_ref, st_ref, dt_ref):
    h = jnp.dot(x_ref[...], w_ref[...], preferred_element_type=jnp.float32)
    asrc = jnp.dot(h, asrcm_ref[...], preferred_element_type=jnp.float32)
    adst = jnp.dot(h, adstm_ref[...], preferred_element_type=jnp.float32)
    z = jnp.zeros((BLK, 12), jnp.float32)
    for c in range(2):
        st_ref[c] = jnp.concatenate(
            [asrc[:, 4 * c:4 * c + 4], z, h[:, 64 * c:64 * c + 64]], axis=1)
    dt_ref[...] = jnp.concatenate([adst, jnp.zeros((BLK, 8), jnp.float32)],
                                  axis=1)


def _prep0(x, W0, Asrc0, Adst0):
    return pl.pallas_call(
        _prep0_body,
        grid=(N // BLK,),
        in_specs=[
            pl.BlockSpec((BLK, 128), lambda i: (i, 0)),
            pl.BlockSpec((128, 128), lambda i: (0, 0)),
            pl.BlockSpec((128, 8), lambda i: (0, 0)),
            pl.BlockSpec((128, 8), lambda i: (0, 0)),
        ],
        out_specs=[
            pl.BlockSpec((2, BLK, 80), lambda i: (0, i, 0)),
            pl.BlockSpec((BLK, 16), lambda i: (i, 0)),
        ],
        out_shape=[
            jax.ShapeDtypeStruct((2, N, 80), jnp.float32),
            jax.ShapeDtypeStruct((N, 16), jnp.float32),
        ],
    )(x, W0, Asrc0, Adst0)


def _mid_body(acc_ref, b0_ref, w1_ref, asrcm_ref, adstm_ref, st_ref, dt_ref):
    cols = []
    for c in range(2):
        for k in range(4):
            wsum = acc_ref[c, :, k:k + 1] + 1e-16
            cols.append(acc_ref[c, :, 16 + 16 * k:32 + 16 * k] / wsum)
    h = jnp.concatenate(cols, axis=1) + b0_ref[...]
    h = jnp.where(h > 0, h, jnp.exp(h) - 1.0)
    h1 = jnp.dot(h, w1_ref[...], preferred_element_type=jnp.float32)
    asrc = jnp.dot(h1, asrcm_ref[...], preferred_element_type=jnp.float32)
    adst = jnp.dot(h1, adstm_ref[...], preferred_element_type=jnp.float32)
    z = jnp.zeros((BLK, 14), jnp.float32)
    for t in range(4):
        hd = 4 * (t // 2) + 2 * (t % 2)
        st_ref[t] = jnp.concatenate(
            [asrc[:, hd:hd + 2], z, h1[:, 40 * hd:40 * hd + 80]], axis=1)
    dt_ref[...] = jnp.concatenate([adst, jnp.zeros((BLK, 8), jnp.float32)],
                                  axis=1)


def _mid(acc0, b0, W1, Asrc1, Adst1):
    return pl.pallas_call(
        _mid_body,
        grid=(N // BLK,),
        in_specs=[
            pl.BlockSpec((2, BLK, 80), lambda i: (0, i, 0)),
            pl.BlockSpec((1, 128), lambda i: (0, 0)),
            pl.BlockSpec((128, 320), lambda i: (0, 0)),
            pl.BlockSpec((320, 8), lambda i: (0, 0)),
            pl.BlockSpec((320, 8), lambda i: (0, 0)),
        ],
        out_specs=[
            pl.BlockSpec((4, BLK, 96), lambda i: (0, i, 0)),
            pl.BlockSpec((BLK, 16), lambda i: (i, 0)),
        ],
        out_shape=[
            jax.ShapeDtypeStruct((4, N, 96), jnp.float32),
            jax.ShapeDtypeStruct((N, 16), jnp.float32),
        ],
    )(acc0, b0, W1, Asrc1, Adst1)


def _final_body(acc0_ref, acc1_ref, b1_ref, out_ref):
    tot = jnp.zeros((BLK, 40), jnp.float32)
    for c in range(2):
        for q, ref in ((0, acc0_ref), (1, acc1_ref)):
            for k in range(2):
                wsum = ref[c, :, k:k + 1] + 1e-16
                tot = tot + ref[c, :, 16 + 40 * k:56 + 40 * k] / wsum
    out_ref[...] = tot * 0.125 + b1_ref[...]


def _final(accq0, accq1, b1):
    return pl.pallas_call(
        _final_body,
        grid=(N // BLK,),
        in_specs=[
            pl.BlockSpec((2, BLK, 96), lambda i: (0, i, 0)),
            pl.BlockSpec((2, BLK, 96), lambda i: (0, i, 0)),
            pl.BlockSpec((1, 40), lambda i: (0, 0)),
        ],
        out_specs=pl.BlockSpec((BLK, 40), lambda i: (i, 0)),
        out_shape=jax.ShapeDtypeStruct((N, 40), jnp.float32),
    )(accq0, accq1, b1)


@functools.lru_cache(maxsize=None)
def _make_sc_edge(R, C, KH, q, tgroups):
    mesh = plsc.VectorSubcoreMesh(core_axis_name="c", subcore_axis_name="s")
    cp = pltpu.CompilerParams()
    if "needs_layout_passes" in pltpu.CompilerParams.__dataclass_fields__:
        cp = dataclasses.replace(cp, needs_layout_passes=False)
    if "use_tc_tiling_on_sc" in pltpu.CompilerParams.__dataclass_fields__:
        cp = dataclasses.replace(cp, use_tc_tiling_on_sc=False)

    @functools.partial(
        pl.kernel,
        out_type=jax.ShapeDtypeStruct((2 * NP, R), jnp.float32),
        mesh=mesh,
        compiler_params=cp,
        scratch_types=[
            pltpu.VMEM((2, EB), jnp.int32),
            pltpu.VMEM((2, EB), jnp.int32),
            pltpu.VMEM((2, EB), jnp.int32),
            pltpu.VMEM((2, EB, R), jnp.float32),
            pltpu.VMEM((2, EB, 16), jnp.float32),
            pltpu.VMEM((2, EB, R), jnp.float32),
            pltpu.VMEM((ZR, R), jnp.float32),
            pltpu.VMEM_SHARED((NP, R), jnp.float32),
        ] + [pltpu.SemaphoreType.DMA] * 10,
    )
    def sc_edge(src_hbm, dst_hbm, st_hbm, dt_hbm, out_hbm,
                sidx, didxg, didxs, srcbuf, dstbuf, msgbuf, zbuf, acc,
                gs0, gs1, gd0, gd1, is0, is1, id0, id1, ss0, ss1):
        c = lax.axis_index("c")
        hd0 = c * 4 + q * KH
        s = lax.axis_index("s")
        coff = (c * (tgroups // 2) + q) * N
        zeros16 = jnp.zeros((16,), jnp.float32)
        iota16 = lax.iota(jnp.int32, 16)
        gsem = (gs0, gs1)
        gdem = (gd0, gd1)
        isem = (is0, is1)
        idem = (id0, id1)
        ssem = (ss0, ss1)

        for j in range(ZR):
            for col0 in range(R // 16):
                zbuf[j, pl.ds(col0 * 16, 16)] = zeros16

        @pl.loop(0, RPS // ZR)
        def _zero(i):
            pltpu.sync_copy(zbuf, acc.at[pl.ds(s * RPS + i * ZR, ZR)])

        plsc.subcore_barrier()

        ebase = s * EP

        def adjust(par):
            for g in range(EB // 16):
                sl = pl.ds(g * 16, 16)
                sidx[par, sl] = sidx[par, sl] + coff

        def issue_gathers(par):
            pltpu.async_copy(st_hbm.at[sidx.at[par]], srcbuf.at[par],
                             gsem[par])
            pltpu.async_copy(dt_hbm.at[didxg.at[par]], dstbuf.at[par],
                             gdem[par])

        def wait_gathers(par):
            pltpu.make_async_copy(st_hbm.at[sidx.at[par]], srcbuf.at[par],
                                  gsem[par]).wait()
            pltpu.make_async_copy(dt_hbm.at[didxg.at[par]], dstbuf.at[par],
                                  gdem[par]).wait()

        def issue_idx(par, blk):
            pltpu.async_copy(src_hbm.at[pl.ds(ebase + blk * EB, EB)],
                             sidx.at[par], isem[par])
            pltpu.async_copy(dst_hbm.at[pl.ds(ebase + blk * EB, EB)],
                             didxg.at[par], idem[par])

        def wait_idx(par, blk):
            pltpu.make_async_copy(src_hbm.at[pl.ds(ebase + blk * EB, EB)],
                                  sidx.at[par], isem[par]).wait()
            pltpu.make_async_copy(dst_hbm.at[pl.ds(ebase + blk * EB, EB)],
                                  didxg.at[par], idem[par]).wait()

        def wait_scatter(par):
            pltpu.make_async_copy(msgbuf.at[par], acc.at[didxs.at[par]],
                                  ssem[par]).wait()

        hdr_mask = iota16 < KH
        align_idx = jnp.bitwise_and(iota16 + hd0, 15)
        nch = (R - 16) // 16
        chunk_heads = []
        for m in range(nch):
            pos = iota16 + (16 * m)
            hsel = jnp.zeros((16,), jnp.int32)
            for t in range(1, KH):
                hsel = hsel + jnp.where(pos >= t * C, 1, 0).astype(jnp.int32)
            chunk_heads.append(hsel)

        def compute(par):
            for j in range(EB):
                arow = srcbuf[par, j, pl.ds(0, 16)]
                drow = dstbuf[par, j, pl.ds(0, 16)]
                adst = drow.at[align_idx].get(mode="promise_in_bounds")
                e = arow + adst
                e = jnp.maximum(e, 0.2 * e)
                w = jnp.where(hdr_mask, jnp.exp(e), 0.0)
                msgbuf[par, j, pl.ds(0, 16)] = w
                for m in range(nch):
                    wv = w.at[chunk_heads[m]].get(mode="promise_in_bounds")
                    hv = srcbuf[par, j, pl.ds(16 + 16 * m, 16)]
                    msgbuf[par, j, pl.ds(16 + 16 * m, 16)] = hv * wv

        pltpu.sync_copy(src_hbm.at[pl.ds(ebase, EB)], sidx.at[0])
        pltpu.sync_copy(dst_hbm.at[pl.ds(ebase, EB)], didxg.at[0])
        adjust(0)
        issue_gathers(0)
        issue_idx(1, 1)

        @pl.loop(0, NB // 2)
        def _pair(i):
            for P in range(2):
                b = 2 * i + P
                Q = 1 - P
                wait_gathers(P)
                if P == 0:
                    wait_idx(Q, b + 1)
                    adjust(Q)
                    issue_gathers(Q)
                else:
                    @pl.when(i < NB // 2 - 1)
                    def _():
                        wait_idx(Q, b + 1)
                        adjust(Q)
                        issue_gathers(Q)

                @pl.when(i > 0)
                def _():
                    wait_scatter(P)
                for g in range(EB // 16):
                    sl = pl.ds(g * 16, 16)
                    didxs[P, sl] = didxg[P, sl]

                @pl.when(i < NB // 2 - 1)
                def _():
                    issue_idx(P, b + 2)

                compute(P)
                pltpu.async_copy(msgbuf.at[P], acc.at[didxs.at[P]], ssem[P],
                                 add=True)

        wait_scatter(0)
        wait_scatter(1)
        plsc.subcore_barrier()

        @pl.loop(0, RPS // ZR)
        def _drain(i):
            row = s * RPS + i * ZR
            pltpu.sync_copy(acc.at[pl.ds(row, ZR)], zbuf)
            pltpu.sync_copy(zbuf, out_hbm.at[pl.ds(c * NP + row, ZR)])

    return sc_edge


def kernel(x, edge_index, W0, a_src0, a_dst0, b0, W1, a_src1, a_dst1, b1):
    src = edge_index[0].astype(jnp.int32)
    dst = edge_index[1].astype(jnp.int32)
    Asrc0 = _head_matrix(a_src0)
    Adst0 = _head_matrix(a_dst0)
    Asrc1 = _head_matrix(a_src1)
    Adst1 = _head_matrix(a_dst1)

    st0, dt0 = _prep0(x, W0, Asrc0, Adst0)
    acc0 = _make_sc_edge(80, 16, 4, 0, 2)(
        src, dst, st0.reshape(2 * N, 80), dt0)
    st1, dt1 = _mid(acc0.reshape(2, NP, 80), b0.reshape(1, 128),
                    W1, Asrc1, Adst1)
    st1f = st1.reshape(4 * N, 96)
    accq0 = _make_sc_edge(96, 40, 2, 0, 4)(src, dst, st1f, dt1)
    accq1 = _make_sc_edge(96, 40, 2, 1, 4)(src, dst, st1f, dt1)
    return _final(accq0.reshape(2, NP, 96), accq1.reshape(2, NP, 96),
                  b1.reshape(1, 40))


# speedup vs baseline: 92.9708x; 1.1670x over previous
import dataclasses
import functools

import jax
import jax.numpy as jnp
from jax import lax
from jax.experimental import pallas as pl
from jax.experimental.pallas import tpu as pltpu
from jax.experimental.pallas import tpu_sc as plsc

N = 10000
NP = 10240
E = 320000
BLK = 200
NSUB = 16
EP = E // NSUB
EB = 80
NB = EP // EB
RPS = NP // NSUB
ZR = 32


def _head_matrix(a):
    H, C = a.shape
    flat = a.reshape(-1)
    eye = jnp.repeat(jnp.eye(H, dtype=a.dtype), C, axis=0)
    return eye * flat[:, None]


def _prep0_body(x_ref, w_ref, asrcm_ref, adstm_ref, st_ref, dt_ref):
    h = jnp.dot(x_ref[...], w_ref[...], preferred_element_type=jnp.float32)
    asrc = jnp.dot(h, asrcm_ref[...], preferred_element_type=jnp.float32)
    adst = jnp.dot(h, adstm_ref[...], preferred_element_type=jnp.float32)
    z = jnp.zeros((BLK, 12), jnp.float32)
    for c in range(2):
        st_ref[c] = jnp.concatenate(
            [asrc[:, 4 * c:4 * c + 4], z, h[:, 64 * c:64 * c + 64]], axis=1)
    dt_ref[...] = jnp.concatenate([adst, jnp.zeros((BLK, 8), jnp.float32)],
                                  axis=1)


def _prep0(x, W0, Asrc0, Adst0):
    return pl.pallas_call(
        _prep0_body,
        grid=(N // BLK,),
        in_specs=[
            pl.BlockSpec((BLK, 128), lambda i: (i, 0)),
            pl.BlockSpec((128, 128), lambda i: (0, 0)),
            pl.BlockSpec((128, 8), lambda i: (0, 0)),
            pl.BlockSpec((128, 8), lambda i: (0, 0)),
        ],
        out_specs=[
            pl.BlockSpec((2, BLK, 80), lambda i: (0, i, 0)),
            pl.BlockSpec((BLK, 16), lambda i: (i, 0)),
        ],
        out_shape=[
            jax.ShapeDtypeStruct((2, N, 80), jnp.float32),
            jax.ShapeDtypeStruct((N, 16), jnp.float32),
        ],
    )(x, W0, Asrc0, Adst0)


def _mid_body(acc_ref, b0_ref, w1_ref, asrcm_ref, adstm_ref, st_ref, dt_ref):
    cols = []
    for c in range(2):
        for k in range(4):
            wsum = acc_ref[c, :, k:k + 1] + 1e-16
            cols.append(acc_ref[c, :, 16 + 16 * k:32 + 16 * k] / wsum)
    h = jnp.concatenate(cols, axis=1) + b0_ref[...]
    h = jnp.where(h > 0, h, jnp.exp(h) - 1.0)
    h1 = jnp.dot(h, w1_ref[...], preferred_element_type=jnp.float32)
    asrc = jnp.dot(h1, asrcm_ref[...], preferred_element_type=jnp.float32)
    adst = jnp.dot(h1, adstm_ref[...], preferred_element_type=jnp.float32)
    z = jnp.zeros((BLK, 14), jnp.float32)
    for t in range(4):
        hd = 4 * (t // 2) + 2 * (t % 2)
        st_ref[t] = jnp.concatenate(
            [asrc[:, hd:hd + 2], z, h1[:, 40 * hd:40 * hd + 80]], axis=1)
    dt_ref[...] = jnp.concatenate([adst, jnp.zeros((BLK, 8), jnp.float32)],
                                  axis=1)


def _mid(acc0, b0, W1, Asrc1, Adst1):
    return pl.pallas_call(
        _mid_body,
        grid=(N // BLK,),
        in_specs=[
            pl.BlockSpec((2, BLK, 80), lambda i: (0, i, 0)),
            pl.BlockSpec((1, 128), lambda i: (0, 0)),
            pl.BlockSpec((128, 320), lambda i: (0, 0)),
            pl.BlockSpec((320, 8), lambda i: (0, 0)),
            pl.BlockSpec((320, 8), lambda i: (0, 0)),
        ],
        out_specs=[
            pl.BlockSpec((4, BLK, 96), lambda i: (0, i, 0)),
            pl.BlockSpec((BLK, 16), lambda i: (i, 0)),
        ],
        out_shape=[
            jax.ShapeDtypeStruct((4, N, 96), jnp.float32),
            jax.ShapeDtypeStruct((N, 16), jnp.float32),
        ],
    )(acc0, b0, W1, Asrc1, Adst1)


def _final_body(acc0_ref, acc1_ref, b1_ref, out_ref):
    tot = jnp.zeros((BLK, 40), jnp.float32)
    for c in range(2):
        for q, ref in ((0, acc0_ref), (1, acc1_ref)):
            for k in range(2):
                wsum = ref[c, :, k:k + 1] + 1e-16
                tot = tot + ref[c, :, 16 + 40 * k:56 + 40 * k] / wsum
    out_ref[...] = tot * 0.125 + b1_ref[...]


def _final(accq0, accq1, b1):
    return pl.pallas_call(
        _final_body,
        grid=(N // BLK,),
        in_specs=[
            pl.BlockSpec((2, BLK, 96), lambda i: (0, i, 0)),
            pl.BlockSpec((2, BLK, 96), lambda i: (0, i, 0)),
            pl.BlockSpec((1, 40), lambda i: (0, 0)),
        ],
        out_specs=pl.BlockSpec((BLK, 40), lambda i: (i, 0)),
        out_shape=jax.ShapeDtypeStruct((N, 40), jnp.float32),
    )(accq0, accq1, b1)


@functools.lru_cache(maxsize=None)
def _make_sc_edge(R, C, KH, q, tgroups):
    mesh = plsc.VectorSubcoreMesh(core_axis_name="c", subcore_axis_name="s")
    cp = pltpu.CompilerParams()
    if "needs_layout_passes" in pltpu.CompilerParams.__dataclass_fields__:
        cp = dataclasses.replace(cp, needs_layout_passes=False)
    if "use_tc_tiling_on_sc" in pltpu.CompilerParams.__dataclass_fields__:
        cp = dataclasses.replace(cp, use_tc_tiling_on_sc=False)

    @functools.partial(
        pl.kernel,
        out_type=jax.ShapeDtypeStruct((2 * NP, R), jnp.float32),
        mesh=mesh,
        compiler_params=cp,
        scratch_types=[
            pltpu.VMEM((2, 2, EB), jnp.int32),
            pltpu.VMEM((2, EB), jnp.int32),
            pltpu.VMEM((2, EB, R), jnp.float32),
            pltpu.VMEM((2, EB, 16), jnp.float32),
            pltpu.VMEM((2, EB, R), jnp.float32),
            pltpu.VMEM((ZR, R), jnp.float32),
            pltpu.VMEM_SHARED((NP, R), jnp.float32),
        ] + [pltpu.SemaphoreType.DMA] * 8,
    )
    def sc_edge(eidx_hbm, st_hbm, dt_hbm, out_hbm,
                eidxb, didxs, srcbuf, dstbuf, msgbuf, zbuf, acc,
                gs0, gs1, gd0, gd1, ie0, ie1, ss0, ss1):
        c = lax.axis_index("c")
        hd0 = c * 4 + q * KH
        s = lax.axis_index("s")
        coff = (c * (tgroups // 2) + q) * N
        zeros16 = jnp.zeros((16,), jnp.float32)
        iota16 = lax.iota(jnp.int32, 16)
        gsem = (gs0, gs1)
        gdem = (gd0, gd1)
        iem = (ie0, ie1)
        ssem = (ss0, ss1)

        for j in range(ZR):
            for col0 in range(R // 16):
                zbuf[j, pl.ds(col0 * 16, 16)] = zeros16

        @pl.loop(0, RPS // ZR)
        def _zero(i):
            pltpu.sync_copy(zbuf, acc.at[pl.ds(s * RPS + i * ZR, ZR)])

        plsc.subcore_barrier()

        ebase = s * EP

        def adjust(par):
            for g in range(EB // 16):
                sl = pl.ds(g * 16, 16)
                eidxb[par, 0, sl] = eidxb[par, 0, sl] + coff

        def issue_gathers(par):
            pltpu.async_copy(st_hbm.at[eidxb.at[par, 0]], srcbuf.at[par],
                             gsem[par])
            pltpu.async_copy(dt_hbm.at[eidxb.at[par, 1]], dstbuf.at[par],
                             gdem[par])

        def wait_gathers(par):
            pltpu.make_async_copy(st_hbm.at[eidxb.at[par, 0]],
                                  srcbuf.at[par], gsem[par]).wait()
            pltpu.make_async_copy(dt_hbm.at[eidxb.at[par, 1]],
                                  dstbuf.at[par], gdem[par]).wait()

        def issue_idx(par, blk):
            pltpu.async_copy(
                eidx_hbm.at[pl.ds(0, 2), pl.ds(ebase + blk * EB, EB)],
                eidxb.at[par], iem[par])

        def wait_idx(par, blk):
            pltpu.make_async_copy(
                eidx_hbm.at[pl.ds(0, 2), pl.ds(ebase + blk * EB, EB)],
                eidxb.at[par], iem[par]).wait()

        def wait_scatter(par):
            pltpu.make_async_copy(msgbuf.at[par], acc.at[didxs.at[par]],
                                  ssem[par]).wait()

        hdr_mask = iota16 < KH
        align_idx = jnp.bitwise_and(iota16 + hd0, 15)
        nch = (R - 16) // 16
        chunk_heads = []
        for m in range(nch):
            pos = iota16 + (16 * m)
            hsel = jnp.zeros((16,), jnp.int32)
            for t in range(1, KH):
                hsel = hsel + jnp.where(pos >= t * C, 1, 0).astype(jnp.int32)
            chunk_heads.append(hsel)

        def compute(par):
            for j in range(EB):
                arow = srcbuf[par, j, pl.ds(0, 16)]
                drow = dstbuf[par, j, pl.ds(0, 16)]
                adst = drow.at[align_idx].get(mode="promise_in_bounds")
                e = arow + adst
                e = jnp.maximum(e, 0.2 * e)
                w = jnp.where(hdr_mask, jnp.exp(e), 0.0)
                msgbuf[par, j, pl.ds(0, 16)] = w
                for m in range(nch):
                    wv = w.at[chunk_heads[m]].get(mode="promise_in_bounds")
                    hv = srcbuf[par, j, pl.ds(16 + 16 * m, 16)]
                    msgbuf[par, j, pl.ds(16 + 16 * m, 16)] = hv * wv

        pltpu.sync_copy(eidx_hbm.at[pl.ds(0, 2), pl.ds(ebase, EB)],
                        eidxb.at[0])
        adjust(0)
        issue_gathers(0)
        issue_idx(1, 1)

        @pl.loop(0, NB // 2)
        def _pair(i):
            for P in range(2):
                b = 2 * i + P
                Q = 1 - P
                if P == 0:
                    wait_idx(Q, b + 1)
                    adjust(Q)
                    issue_gathers(Q)
                else:
                    @pl.when(i < NB // 2 - 1)
                    def _():
                        wait_idx(Q, b + 1)
                        adjust(Q)
                        issue_gathers(Q)

                wait_gathers(P)

                @pl.when(i > 0)
                def _():
                    wait_scatter(P)
                for g in range(EB // 16):
                    sl = pl.ds(g * 16, 16)
                    didxs[P, sl] = eidxb[P, 1, sl]

                @pl.when(i < NB // 2 - 1)
                def _():
                    issue_idx(P, b + 2)

                compute(P)
                pltpu.async_copy(msgbuf.at[P], acc.at[didxs.at[P]], ssem[P],
                                 add=True)

        wait_scatter(0)
        wait_scatter(1)
        plsc.subcore_barrier()

        @pl.loop(0, RPS // ZR)
        def _drain(i):
            row = s * RPS + i * ZR
            pltpu.sync_copy(acc.at[pl.ds(row, ZR)], zbuf)
            pltpu.sync_copy(zbuf, out_hbm.at[pl.ds(c * NP + row, ZR)])

    return sc_edge


def kernel(x, edge_index, W0, a_src0, a_dst0, b0, W1, a_src1, a_dst1, b1):
    eidx = edge_index.astype(jnp.int32)
    Asrc0 = _head_matrix(a_src0)
    Adst0 = _head_matrix(a_dst0)
    Asrc1 = _head_matrix(a_src1)
    Adst1 = _head_matrix(a_dst1)

    st0, dt0 = _prep0(x, W0, Asrc0, Adst0)
    acc0 = _make_sc_edge(80, 16, 4, 0, 2)(
        eidx, st0.reshape(2 * N, 80), dt0)
    st1, dt1 = _mid(acc0.reshape(2, NP, 80), b0.reshape(1, 128),
                    W1, Asrc1, Adst1)
    st1f = st1.reshape(4 * N, 96)
    accq0 = _make_sc_edge(96, 40, 2, 0, 4)(eidx, st1f, dt1)
    accq1 = _make_sc_edge(96, 40, 2, 1, 4)(eidx, st1f, dt1)
    return _final(accq0.reshape(2, NP, 96), accq1.reshape(2, NP, 96),
                  b1.reshape(1, 40))


# speedup vs baseline: 99.8436x; 1.0739x over previous
import dataclasses
import functools

import jax
import jax.numpy as jnp
from jax import lax
from jax.experimental import pallas as pl
from jax.experimental.pallas import tpu as pltpu
from jax.experimental.pallas import tpu_sc as plsc

N = 10000
NP = 10240
E = 320000
BLK = 1000
NSUB = 16
EP = E // NSUB
EB = 80
NB = EP // EB
RPS = NP // NSUB
ZR = 32


def _head_matrix(a):
    H, C = a.shape
    flat = a.reshape(-1)
    eye = jnp.repeat(jnp.eye(H, dtype=a.dtype), C, axis=0)
    return eye * flat[:, None]


def _prep0_body(x_ref, w_ref, asrcm_ref, adstm_ref, st_ref, dt_ref):
    h = jnp.dot(x_ref[...], w_ref[...], preferred_element_type=jnp.float32)
    asrc = jnp.dot(h, asrcm_ref[...], preferred_element_type=jnp.float32)
    adst = jnp.dot(h, adstm_ref[...], preferred_element_type=jnp.float32)
    z = jnp.zeros((BLK, 12), jnp.float32)
    for c in range(2):
        st_ref[c] = jnp.concatenate(
            [asrc[:, 4 * c:4 * c + 4], z, h[:, 64 * c:64 * c + 64]], axis=1)
    dt_ref[...] = jnp.concatenate([adst, jnp.zeros((BLK, 8), jnp.float32)],
                                  axis=1)


def _prep0(x, W0, Asrc0, Adst0):
    return pl.pallas_call(
        _prep0_body,
        grid=(N // BLK,),
        in_specs=[
            pl.BlockSpec((BLK, 128), lambda i: (i, 0)),
            pl.BlockSpec((128, 128), lambda i: (0, 0)),
            pl.BlockSpec((128, 8), lambda i: (0, 0)),
            pl.BlockSpec((128, 8), lambda i: (0, 0)),
        ],
        out_specs=[
            pl.BlockSpec((2, BLK, 80), lambda i: (0, i, 0)),
            pl.BlockSpec((BLK, 16), lambda i: (i, 0)),
        ],
        out_shape=[
            jax.ShapeDtypeStruct((2, N, 80), jnp.float32),
            jax.ShapeDtypeStruct((N, 16), jnp.float32),
        ],
    )(x, W0, Asrc0, Adst0)


def _mid_body(acc_ref, b0_ref, w1_ref, asrcm_ref, adstm_ref, st_ref, dt_ref):
    cols = []
    for c in range(2):
        for k in range(4):
            rec = 1.0 / (acc_ref[c, :, k:k + 1] + 1e-16)
            cols.append(acc_ref[c, :, 16 + 16 * k:32 + 16 * k] * rec)
    h = jnp.concatenate(cols, axis=1) + b0_ref[...]
    h = jnp.where(h > 0, h, jnp.exp(h) - 1.0)
    h1 = jnp.dot(h, w1_ref[...], preferred_element_type=jnp.float32)
    asrc = jnp.dot(h1, asrcm_ref[...], preferred_element_type=jnp.float32)
    adst = jnp.dot(h1, adstm_ref[...], preferred_element_type=jnp.float32)
    z = jnp.zeros((BLK, 14), jnp.float32)
    for t in range(4):
        hd = 4 * (t // 2) + 2 * (t % 2)
        st_ref[t] = jnp.concatenate(
            [asrc[:, hd:hd + 2], z, h1[:, 40 * hd:40 * hd + 80]], axis=1)
    dt_ref[...] = jnp.concatenate([adst, jnp.zeros((BLK, 8), jnp.float32)],
                                  axis=1)


def _mid(acc0, b0, W1, Asrc1, Adst1):
    return pl.pallas_call(
        _mid_body,
        grid=(N // BLK,),
        in_specs=[
            pl.BlockSpec((2, BLK, 80), lambda i: (0, i, 0)),
            pl.BlockSpec((1, 128), lambda i: (0, 0)),
            pl.BlockSpec((128, 320), lambda i: (0, 0)),
            pl.BlockSpec((320, 8), lambda i: (0, 0)),
            pl.BlockSpec((320, 8), lambda i: (0, 0)),
        ],
        out_specs=[
            pl.BlockSpec((4, BLK, 96), lambda i: (0, i, 0)),
            pl.BlockSpec((BLK, 16), lambda i: (i, 0)),
        ],
        out_shape=[
            jax.ShapeDtypeStruct((4, N, 96), jnp.float32),
            jax.ShapeDtypeStruct((N, 16), jnp.float32),
        ],
    )(acc0, b0, W1, Asrc1, Adst1)


def _final_body(acc0_ref, acc1_ref, b1_ref, out_ref):
    tot = jnp.zeros((BLK, 40), jnp.float32)
    for c in range(2):
        for q, ref in ((0, acc0_ref), (1, acc1_ref)):
            for k in range(2):
                rec = 1.0 / (ref[c, :, k:k + 1] + 1e-16)
                tot = tot + ref[c, :, 16 + 40 * k:56 + 40 * k] * rec
    out_ref[...] = tot * 0.125 + b1_ref[...]


def _final(accq0, accq1, b1):
    return pl.pallas_call(
        _final_body,
        grid=(N // BLK,),
        in_specs=[
            pl.BlockSpec((2, BLK, 96), lambda i: (0, i, 0)),
            pl.BlockSpec((2, BLK, 96), lambda i: (0, i, 0)),
            pl.BlockSpec((1, 40), lambda i: (0, 0)),
        ],
        out_specs=pl.BlockSpec((BLK, 40), lambda i: (i, 0)),
        out_shape=jax.ShapeDtypeStruct((N, 40), jnp.float32),
    )(accq0, accq1, b1)


@functools.lru_cache(maxsize=None)
def _make_sc_edge(R, C, KH, q, tgroups):
    mesh = plsc.VectorSubcoreMesh(core_axis_name="c", subcore_axis_name="s")
    cp = pltpu.CompilerParams()
    if "needs_layout_passes" in pltpu.CompilerParams.__dataclass_fields__:
        cp = dataclasses.replace(cp, needs_layout_passes=False)
    if "use_tc_tiling_on_sc" in pltpu.CompilerParams.__dataclass_fields__:
        cp = dataclasses.replace(cp, use_tc_tiling_on_sc=False)

    @functools.partial(
        pl.kernel,
        out_type=jax.ShapeDtypeStruct((2 * NP, R), jnp.float32),
        mesh=mesh,
        compiler_params=cp,
        scratch_types=[
            pltpu.VMEM((2, 2, EB), jnp.int32),
            pltpu.VMEM((2, EB), jnp.int32),
            pltpu.VMEM((2, EB, R), jnp.float32),
            pltpu.VMEM((2, EB, 16), jnp.float32),
            pltpu.VMEM((2, EB, R), jnp.float32),
            pltpu.VMEM((ZR, R), jnp.float32),
            pltpu.VMEM_SHARED((NP, R), jnp.float32),
        ] + [pltpu.SemaphoreType.DMA] * 8,
    )
    def sc_edge(eidx_hbm, st_hbm, dt_hbm, out_hbm,
                eidxb, didxs, srcbuf, dstbuf, msgbuf, zbuf, acc,
                gs0, gs1, gd0, gd1, ie0, ie1, ss0, ss1):
        c = lax.axis_index("c")
        hd0 = c * 4 + q * KH
        s = lax.axis_index("s")
        coff = (c * (tgroups // 2) + q) * N
        zeros16 = jnp.zeros((16,), jnp.float32)
        iota16 = lax.iota(jnp.int32, 16)
        gsem = (gs0, gs1)
        gdem = (gd0, gd1)
        iem = (ie0, ie1)
        ssem = (ss0, ss1)

        for j in range(ZR):
            for col0 in range(R // 16):
                zbuf[j, pl.ds(col0 * 16, 16)] = zeros16

        @pl.loop(0, RPS // ZR)
        def _zero(i):
            pltpu.sync_copy(zbuf, acc.at[pl.ds(s * RPS + i * ZR, ZR)])

        plsc.subcore_barrier()

        ebase = s * EP

        def adjust(par):
            for g in range(EB // 16):
                sl = pl.ds(g * 16, 16)
                eidxb[par, 0, sl] = eidxb[par, 0, sl] + coff

        def issue_gathers(par):
            pltpu.async_copy(st_hbm.at[eidxb.at[par, 0]], srcbuf.at[par],
                             gsem[par])
            pltpu.async_copy(dt_hbm.at[eidxb.at[par, 1]], dstbuf.at[par],
                             gdem[par])

        def wait_gathers(par):
            pltpu.make_async_copy(st_hbm.at[eidxb.at[par, 0]],
                                  srcbuf.at[par], gsem[par]).wait()
            pltpu.make_async_copy(dt_hbm.at[eidxb.at[par, 1]],
                                  dstbuf.at[par], gdem[par]).wait()

        def issue_idx(par, blk):
            pltpu.async_copy(
                eidx_hbm.at[pl.ds(0, 2), pl.ds(ebase + blk * EB, EB)],
                eidxb.at[par], iem[par])

        def wait_idx(par, blk):
            pltpu.make_async_copy(
                eidx_hbm.at[pl.ds(0, 2), pl.ds(ebase + blk * EB, EB)],
                eidxb.at[par], iem[par]).wait()

        def wait_scatter(par):
            pltpu.make_async_copy(msgbuf.at[par], acc.at[didxs.at[par]],
                                  ssem[par]).wait()

        hdr_mask = iota16 < KH
        align_idx = jnp.bitwise_and(iota16 + hd0, 15)
        nch = (R - 16) // 16
        chunk_heads = []
        for m in range(nch):
            pos = iota16 + (16 * m)
            hsel = jnp.zeros((16,), jnp.int32)
            for t in range(1, KH):
                hsel = hsel + jnp.where(pos >= t * C, 1, 0).astype(jnp.int32)
            chunk_heads.append(hsel)

        def compute(par):
            for j in range(EB):
                arow = srcbuf[par, j, pl.ds(0, 16)]
                drow = dstbuf[par, j, pl.ds(0, 16)]
                adst = drow.at[align_idx].get(mode="promise_in_bounds")
                e = arow + adst
                e = jnp.maximum(e, 0.2 * e)
                w = jnp.where(hdr_mask, jnp.exp(e), 0.0)
                msgbuf[par, j, pl.ds(0, 16)] = w
                for m in range(nch):
                    wv = w.at[chunk_heads[m]].get(mode="promise_in_bounds")
                    hv = srcbuf[par, j, pl.ds(16 + 16 * m, 16)]
                    msgbuf[par, j, pl.ds(16 + 16 * m, 16)] = hv * wv

        pltpu.sync_copy(eidx_hbm.at[pl.ds(0, 2), pl.ds(ebase, EB)],
                        eidxb.at[0])
        adjust(0)
        issue_gathers(0)
        issue_idx(1, 1)

        @pl.loop(0, NB // 2)
        def _pair(i):
            for P in range(2):
                b = 2 * i + P
                Q = 1 - P
                if P == 0:
                    wait_idx(Q, b + 1)
                    adjust(Q)
                    issue_gathers(Q)
                else:
                    @pl.when(i < NB // 2 - 1)
                    def _():
                        wait_idx(Q, b + 1)
                        adjust(Q)
                        issue_gathers(Q)

                wait_gathers(P)

                @pl.when(i > 0)
                def _():
                    wait_scatter(P)
                for g in range(EB // 16):
                    sl = pl.ds(g * 16, 16)
                    didxs[P, sl] = eidxb[P, 1, sl]

                @pl.when(i < NB // 2 - 1)
                def _():
                    issue_idx(P, b + 2)

                compute(P)
                pltpu.async_copy(msgbuf.at[P], acc.at[didxs.at[P]], ssem[P],
                                 add=True)

        wait_scatter(0)
        wait_scatter(1)
        plsc.subcore_barrier()

        @pl.loop(0, RPS // ZR)
        def _drain(i):
            row = s * RPS + i * ZR
            pltpu.sync_copy(acc.at[pl.ds(row, ZR)], zbuf)
            pltpu.sync_copy(zbuf, out_hbm.at[pl.ds(c * NP + row, ZR)])

    return sc_edge


def kernel(x, edge_index, W0, a_src0, a_dst0, b0, W1, a_src1, a_dst1, b1):
    eidx = edge_index.astype(jnp.int32)
    Asrc0 = _head_matrix(a_src0)
    Adst0 = _head_matrix(a_dst0)
    Asrc1 = _head_matrix(a_src1)
    Adst1 = _head_matrix(a_dst1)

    st0, dt0 = _prep0(x, W0, Asrc0, Adst0)
    acc0 = _make_sc_edge(80, 16, 4, 0, 2)(
        eidx, st0.reshape(2 * N, 80), dt0)
    st1, dt1 = _mid(acc0.reshape(2, NP, 80), b0.reshape(1, 128),
                    W1, Asrc1, Adst1)
    st1f = st1.reshape(4 * N, 96)
    accq0 = _make_sc_edge(96, 40, 2, 0, 4)(eidx, st1f, dt1)
    accq1 = _make_sc_edge(96, 40, 2, 1, 4)(eidx, st1f, dt1)
    return _final(accq0.reshape(2, NP, 96), accq1.reshape(2, NP, 96),
                  b1.reshape(1, 40))


# speedup vs baseline: 100.6457x; 1.0080x over previous
import dataclasses
import functools

import jax
import jax.numpy as jnp
from jax import lax
from jax.experimental import pallas as pl
from jax.experimental.pallas import tpu as pltpu
from jax.experimental.pallas import tpu_sc as plsc

N = 10000
NP = 10240
E = 320000
BLK = 1000
NSUB = 16
EP = E // NSUB
EB = 80
NB = EP // EB
RPS = NP // NSUB
ZR = 64


def _head_matrix(a):
    H, C = a.shape
    flat = a.reshape(-1)
    eye = jnp.repeat(jnp.eye(H, dtype=a.dtype), C, axis=0)
    return eye * flat[:, None]


def _prep0_body(x_ref, w_ref, asrcm_ref, adstm_ref, st_ref, dt_ref):
    h = jnp.dot(x_ref[...], w_ref[...], preferred_element_type=jnp.float32)
    asrc = jnp.dot(h, asrcm_ref[...], preferred_element_type=jnp.float32)
    adst = jnp.dot(h, adstm_ref[...], preferred_element_type=jnp.float32)
    z = jnp.zeros((BLK, 12), jnp.float32)
    for c in range(2):
        st_ref[c] = jnp.concatenate(
            [asrc[:, 4 * c:4 * c + 4], z, h[:, 64 * c:64 * c + 64]], axis=1)
    dt_ref[...] = jnp.concatenate([adst, jnp.zeros((BLK, 8), jnp.float32)],
                                  axis=1)


def _prep0(x, W0, Asrc0, Adst0):
    return pl.pallas_call(
        _prep0_body,
        grid=(N // BLK,),
        in_specs=[
            pl.BlockSpec((BLK, 128), lambda i: (i, 0)),
            pl.BlockSpec((128, 128), lambda i: (0, 0)),
            pl.BlockSpec((128, 8), lambda i: (0, 0)),
            pl.BlockSpec((128, 8), lambda i: (0, 0)),
        ],
        out_specs=[
            pl.BlockSpec((2, BLK, 80), lambda i: (0, i, 0)),
            pl.BlockSpec((BLK, 16), lambda i: (i, 0)),
        ],
        out_shape=[
            jax.ShapeDtypeStruct((2, N, 80), jnp.float32),
            jax.ShapeDtypeStruct((N, 16), jnp.float32),
        ],
    )(x, W0, Asrc0, Adst0)


def _mid_body(acc_ref, b0_ref, w1_ref, asrcm_ref, adstm_ref, st_ref, dt_ref):
    cols = []
    for c in range(2):
        for k in range(4):
            rec = 1.0 / (acc_ref[c, :, k:k + 1] + 1e-16)
            cols.append(acc_ref[c, :, 16 + 16 * k:32 + 16 * k] * rec)
    h = jnp.concatenate(cols, axis=1) + b0_ref[...]
    h = jnp.where(h > 0, h, jnp.exp(h) - 1.0)
    h1 = jnp.dot(h, w1_ref[...], preferred_element_type=jnp.float32)
    asrc = jnp.dot(h1, asrcm_ref[...], preferred_element_type=jnp.float32)
    adst = jnp.dot(h1, adstm_ref[...], preferred_element_type=jnp.float32)
    z = jnp.zeros((BLK, 14), jnp.float32)
    for t in range(4):
        hd = 4 * (t // 2) + 2 * (t % 2)
        st_ref[t] = jnp.concatenate(
            [asrc[:, hd:hd + 2], z, h1[:, 40 * hd:40 * hd + 80]], axis=1)
    dt_ref[...] = jnp.concatenate([adst, jnp.zeros((BLK, 8), jnp.float32)],
                                  axis=1)


def _mid(acc0, b0, W1, Asrc1, Adst1):
    return pl.pallas_call(
        _mid_body,
        grid=(N // BLK,),
        in_specs=[
            pl.BlockSpec((2, BLK, 80), lambda i: (0, i, 0)),
            pl.BlockSpec((1, 128), lambda i: (0, 0)),
            pl.BlockSpec((128, 320), lambda i: (0, 0)),
            pl.BlockSpec((320, 8), lambda i: (0, 0)),
            pl.BlockSpec((320, 8), lambda i: (0, 0)),
        ],
        out_specs=[
            pl.BlockSpec((4, BLK, 96), lambda i: (0, i, 0)),
            pl.BlockSpec((BLK, 16), lambda i: (i, 0)),
        ],
        out_shape=[
            jax.ShapeDtypeStruct((4, N, 96), jnp.float32),
            jax.ShapeDtypeStruct((N, 16), jnp.float32),
        ],
    )(acc0, b0, W1, Asrc1, Adst1)


def _final_body(acc0_ref, acc1_ref, b1_ref, out_ref):
    tot = jnp.zeros((BLK, 40), jnp.float32)
    for c in range(2):
        for q, ref in ((0, acc0_ref), (1, acc1_ref)):
            for k in range(2):
                rec = 1.0 / (ref[c, :, k:k + 1] + 1e-16)
                tot = tot + ref[c, :, 16 + 40 * k:56 + 40 * k] * rec
    out_ref[...] = tot * 0.125 + b1_ref[...]


def _final(accq0, accq1, b1):
    return pl.pallas_call(
        _final_body,
        grid=(N // BLK,),
        in_specs=[
            pl.BlockSpec((2, BLK, 96), lambda i: (0, i, 0)),
            pl.BlockSpec((2, BLK, 96), lambda i: (0, i, 0)),
            pl.BlockSpec((1, 40), lambda i: (0, 0)),
        ],
        out_specs=pl.BlockSpec((BLK, 40), lambda i: (i, 0)),
        out_shape=jax.ShapeDtypeStruct((N, 40), jnp.float32),
    )(accq0, accq1, b1)


@functools.lru_cache(maxsize=None)
def _make_sc_edge(R, C, KH, q, tgroups):
    mesh = plsc.VectorSubcoreMesh(core_axis_name="c", subcore_axis_name="s")
    cp = pltpu.CompilerParams()
    if "needs_layout_passes" in pltpu.CompilerParams.__dataclass_fields__:
        cp = dataclasses.replace(cp, needs_layout_passes=False)
    if "use_tc_tiling_on_sc" in pltpu.CompilerParams.__dataclass_fields__:
        cp = dataclasses.replace(cp, use_tc_tiling_on_sc=False)

    @functools.partial(
        pl.kernel,
        out_type=jax.ShapeDtypeStruct((2 * NP, R), jnp.float32),
        mesh=mesh,
        compiler_params=cp,
        scratch_types=[
            pltpu.VMEM((2, 2, EB), jnp.int32),
            pltpu.VMEM((2, EB), jnp.int32),
            pltpu.VMEM((2, EB, R), jnp.float32),
            pltpu.VMEM((2, EB, 16), jnp.float32),
            pltpu.VMEM((2, EB, R), jnp.float32),
            pltpu.VMEM((ZR, R), jnp.float32),
            pltpu.VMEM_SHARED((NP, R), jnp.float32),
        ] + [pltpu.SemaphoreType.DMA] * 8,
    )
    def sc_edge(eidx_hbm, st_hbm, dt_hbm, out_hbm,
                eidxb, didxs, srcbuf, dstbuf, msgbuf, zbuf, acc,
                gs0, gs1, gd0, gd1, ie0, ie1, ss0, ss1):
        c = lax.axis_index("c")
        hd0 = c * 4 + q * KH
        s = lax.axis_index("s")
        coff = (c * (tgroups // 2) + q) * N
        zeros16 = jnp.zeros((16,), jnp.float32)
        iota16 = lax.iota(jnp.int32, 16)
        gsem = (gs0, gs1)
        gdem = (gd0, gd1)
        iem = (ie0, ie1)
        ssem = (ss0, ss1)

        for j in range(ZR):
            for col0 in range(R // 16):
                zbuf[j, pl.ds(col0 * 16, 16)] = zeros16

        @pl.loop(0, RPS // ZR)
        def _zero(i):
            pltpu.sync_copy(zbuf, acc.at[pl.ds(s * RPS + i * ZR, ZR)])

        plsc.subcore_barrier()

        ebase = s * EP

        def adjust(par):
            for g in range(EB // 16):
                sl = pl.ds(g * 16, 16)
                eidxb[par, 0, sl] = eidxb[par, 0, sl] + coff

        def issue_gathers(par):
            pltpu.async_copy(st_hbm.at[eidxb.at[par, 0]], srcbuf.at[par],
                             gsem[par])
            pltpu.async_copy(dt_hbm.at[eidxb.at[par, 1]], dstbuf.at[par],
                             gdem[par])

        def wait_gathers(par):
            pltpu.make_async_copy(st_hbm.at[eidxb.at[par, 0]],
                                  srcbuf.at[par], gsem[par]).wait()
            pltpu.make_async_copy(dt_hbm.at[eidxb.at[par, 1]],
                                  dstbuf.at[par], gdem[par]).wait()

        def issue_idx(par, blk):
            pltpu.async_copy(
                eidx_hbm.at[pl.ds(0, 2), pl.ds(ebase + blk * EB, EB)],
                eidxb.at[par], iem[par])

        def wait_idx(par, blk):
            pltpu.make_async_copy(
                eidx_hbm.at[pl.ds(0, 2), pl.ds(ebase + blk * EB, EB)],
                eidxb.at[par], iem[par]).wait()

        def wait_scatter(par):
            pltpu.make_async_copy(msgbuf.at[par], acc.at[didxs.at[par]],
                                  ssem[par]).wait()

        hdr_mask = iota16 < KH
        align_idx = jnp.bitwise_and(iota16 + hd0, 15)
        nch = (R - 16) // 16
        chunk_heads = []
        for m in range(nch):
            pos = iota16 + (16 * m)
            hsel = jnp.zeros((16,), jnp.int32)
            for t in range(1, KH):
                hsel = hsel + jnp.where(pos >= t * C, 1, 0).astype(jnp.int32)
            chunk_heads.append(hsel)

        def compute(par):
            for j in range(EB):
                arow = srcbuf[par, j, pl.ds(0, 16)]
                drow = dstbuf[par, j, pl.ds(0, 16)]
                adst = drow.at[align_idx].get(mode="promise_in_bounds")
                e = arow + adst
                e = jnp.maximum(e, 0.2 * e)
                w = jnp.where(hdr_mask, jnp.exp(e), 0.0)
                msgbuf[par, j, pl.ds(0, 16)] = w
                for m in range(nch):
                    wv = w.at[chunk_heads[m]].get(mode="promise_in_bounds")
                    hv = srcbuf[par, j, pl.ds(16 + 16 * m, 16)]
                    msgbuf[par, j, pl.ds(16 + 16 * m, 16)] = hv * wv

        pltpu.sync_copy(eidx_hbm.at[pl.ds(0, 2), pl.ds(ebase, EB)],
                        eidxb.at[0])
        adjust(0)
        issue_gathers(0)
        issue_idx(1, 1)

        @pl.loop(0, NB // 2)
        def _pair(i):
            for P in range(2):
                b = 2 * i + P
                Q = 1 - P
                if P == 0:
                    wait_idx(Q, b + 1)
                    adjust(Q)
                    issue_gathers(Q)
                else:
                    @pl.when(i < NB // 2 - 1)
                    def _():
                        wait_idx(Q, b + 1)
                        adjust(Q)
                        issue_gathers(Q)

                wait_gathers(P)

                @pl.when(i > 0)
                def _():
                    wait_scatter(P)
                for g in range(EB // 16):
                    sl = pl.ds(g * 16, 16)
                    didxs[P, sl] = eidxb[P, 1, sl]

                @pl.when(i < NB // 2 - 1)
                def _():
                    issue_idx(P, b + 2)

                compute(P)
                pltpu.async_copy(msgbuf.at[P], acc.at[didxs.at[P]], ssem[P],
                                 add=True)

        wait_scatter(0)
        wait_scatter(1)
        plsc.subcore_barrier()

        row = s * RPS
        pltpu.sync_copy(acc.at[pl.ds(row, RPS)],
                        out_hbm.at[pl.ds(c * NP + row, RPS)])

    return sc_edge


def kernel(x, edge_index, W0, a_src0, a_dst0, b0, W1, a_src1, a_dst1, b1):
    eidx = edge_index.astype(jnp.int32)
    Asrc0 = _head_matrix(a_src0)
    Adst0 = _head_matrix(a_dst0)
    Asrc1 = _head_matrix(a_src1)
    Adst1 = _head_matrix(a_dst1)

    st0, dt0 = _prep0(x, W0, Asrc0, Adst0)
    acc0 = _make_sc_edge(80, 16, 4, 0, 2)(
        eidx, st0.reshape(2 * N, 80), dt0)
    st1, dt1 = _mid(acc0.reshape(2, NP, 80), b0.reshape(1, 128),
                    W1, Asrc1, Adst1)
    st1f = st1.reshape(4 * N, 96)
    accq0 = _make_sc_edge(96, 40, 2, 0, 4)(eidx, st1f, dt1)
    accq1 = _make_sc_edge(96, 40, 2, 1, 4)(eidx, st1f, dt1)
    return _final(accq0.reshape(2, NP, 96), accq1.reshape(2, NP, 96),
                  b1.reshape(1, 40))


# speedup vs baseline: 100.8518x; 1.0020x over previous
import dataclasses
import functools

import jax
import jax.numpy as jnp
from jax import lax
from jax.experimental import pallas as pl
from jax.experimental.pallas import tpu as pltpu
from jax.experimental.pallas import tpu_sc as plsc

N = 10000
NP = 10240
E = 320000
BLK = 1000
NSUB = 16
EP = E // NSUB
EB = 80
NB = EP // EB
RPS = NP // NSUB
ZR = 64


def _head_matrix(a):
    H, C = a.shape
    flat = a.reshape(-1)
    eye = jnp.repeat(jnp.eye(H, dtype=a.dtype), C, axis=0)
    return eye * flat[:, None]


def _prep0_body(x_ref, w_ref, asrcm_ref, adstm_ref, st_ref, dt_ref):
    h = jnp.dot(x_ref[...], w_ref[...], preferred_element_type=jnp.float32)
    asrc = jnp.dot(h, asrcm_ref[...], preferred_element_type=jnp.float32)
    adst = jnp.dot(h, adstm_ref[...], preferred_element_type=jnp.float32)
    z = jnp.zeros((BLK, 12), jnp.float32)
    for c in range(2):
        st_ref[c] = jnp.concatenate(
            [asrc[:, 4 * c:4 * c + 4], z, h[:, 64 * c:64 * c + 64]], axis=1)
    dt_ref[...] = jnp.concatenate([adst, jnp.zeros((BLK, 8), jnp.float32)],
                                  axis=1)


def _prep0(x, W0, Asrc0, Adst0):
    return pl.pallas_call(
        _prep0_body,
        grid=(N // BLK,),
        in_specs=[
            pl.BlockSpec((BLK, 128), lambda i: (i, 0)),
            pl.BlockSpec((128, 128), lambda i: (0, 0)),
            pl.BlockSpec((128, 8), lambda i: (0, 0)),
            pl.BlockSpec((128, 8), lambda i: (0, 0)),
        ],
        out_specs=[
            pl.BlockSpec((2, BLK, 80), lambda i: (0, i, 0)),
            pl.BlockSpec((BLK, 16), lambda i: (i, 0)),
        ],
        out_shape=[
            jax.ShapeDtypeStruct((2, N, 80), jnp.float32),
            jax.ShapeDtypeStruct((N, 16), jnp.float32),
        ],
    )(x, W0, Asrc0, Adst0)


def _mid_body(acc_ref, b0_ref, w1_ref, asrcm_ref, adstm_ref, st_ref, dt_ref):
    cols = []
    for c in range(2):
        for k in range(4):
            rec = 1.0 / (acc_ref[c, :, k:k + 1] + 1e-16)
            cols.append(acc_ref[c, :, 16 + 16 * k:32 + 16 * k] * rec)
    h = jnp.concatenate(cols, axis=1) + b0_ref[...]
    h = jnp.where(h > 0, h, jnp.exp(h) - 1.0)
    h1 = jnp.dot(h, w1_ref[...], preferred_element_type=jnp.float32)
    asrc = jnp.dot(h1, asrcm_ref[...], preferred_element_type=jnp.float32)
    adst = jnp.dot(h1, adstm_ref[...], preferred_element_type=jnp.float32)
    z = jnp.zeros((BLK, 14), jnp.float32)
    for t in range(4):
        hd = 4 * (t // 2) + 2 * (t % 2)
        st_ref[t] = jnp.concatenate(
            [asrc[:, hd:hd + 2], z, h1[:, 40 * hd:40 * hd + 80]], axis=1)
    dt_ref[...] = jnp.concatenate([adst, jnp.zeros((BLK, 8), jnp.float32)],
                                  axis=1)


def _mid(acc0, b0, W1, Asrc1, Adst1):
    return pl.pallas_call(
        _mid_body,
        grid=(N // BLK,),
        in_specs=[
            pl.BlockSpec((2, BLK, 80), lambda i: (0, i, 0)),
            pl.BlockSpec((1, 128), lambda i: (0, 0)),
            pl.BlockSpec((128, 320), lambda i: (0, 0)),
            pl.BlockSpec((320, 8), lambda i: (0, 0)),
            pl.BlockSpec((320, 8), lambda i: (0, 0)),
        ],
        out_specs=[
            pl.BlockSpec((4, BLK, 96), lambda i: (0, i, 0)),
            pl.BlockSpec((BLK, 16), lambda i: (i, 0)),
        ],
        out_shape=[
            jax.ShapeDtypeStruct((4, N, 96), jnp.float32),
            jax.ShapeDtypeStruct((N, 16), jnp.float32),
        ],
    )(acc0, b0, W1, Asrc1, Adst1)


def _final_body(acc0_ref, acc1_ref, b1_ref, out_ref):
    tot = jnp.zeros((BLK, 40), jnp.float32)
    for c in range(2):
        for q, ref in ((0, acc0_ref), (1, acc1_ref)):
            for k in range(2):
                rec = 1.0 / (ref[c, :, k:k + 1] + 1e-16)
                tot = tot + ref[c, :, 16 + 40 * k:56 + 40 * k] * rec
    out_ref[...] = tot * 0.125 + b1_ref[...]


def _final(accq0, accq1, b1):
    return pl.pallas_call(
        _final_body,
        grid=(N // BLK,),
        in_specs=[
            pl.BlockSpec((2, BLK, 96), lambda i: (0, i, 0)),
            pl.BlockSpec((2, BLK, 96), lambda i: (0, i, 0)),
            pl.BlockSpec((1, 40), lambda i: (0, 0)),
        ],
        out_specs=pl.BlockSpec((BLK, 40), lambda i: (i, 0)),
        out_shape=jax.ShapeDtypeStruct((N, 40), jnp.float32),
    )(accq0, accq1, b1)


@functools.lru_cache(maxsize=None)
def _make_sc_edge(R, C, KH, q, tgroups):
    mesh = plsc.VectorSubcoreMesh(core_axis_name="c", subcore_axis_name="s")
    cp = pltpu.CompilerParams()
    if "needs_layout_passes" in pltpu.CompilerParams.__dataclass_fields__:
        cp = dataclasses.replace(cp, needs_layout_passes=False)
    if "use_tc_tiling_on_sc" in pltpu.CompilerParams.__dataclass_fields__:
        cp = dataclasses.replace(cp, use_tc_tiling_on_sc=False)

    @functools.partial(
        pl.kernel,
        out_type=jax.ShapeDtypeStruct((2 * NP, R), jnp.float32),
        mesh=mesh,
        compiler_params=cp,
        scratch_types=[
            pltpu.VMEM((2, 2, EB), jnp.int32),
            pltpu.VMEM((2, EB), jnp.int32),
            pltpu.VMEM((2, EB, R), jnp.float32),
            pltpu.VMEM((2, EB, 16), jnp.float32),
            pltpu.VMEM((2, EB, R), jnp.float32),
            pltpu.VMEM((ZR, R), jnp.float32),
            pltpu.VMEM_SHARED((NP, R), jnp.float32),
        ] + [pltpu.SemaphoreType.DMA] * 8,
    )
    def sc_edge(eidx_hbm, st_hbm, dt_hbm, out_hbm,
                eidxb, didxs, srcbuf, dstbuf, msgbuf, zbuf, acc,
                gs0, gs1, gd0, gd1, ie0, ie1, ss0, ss1):
        c = lax.axis_index("c")
        hd0 = c * 4 + q * KH
        s = lax.axis_index("s")
        coff = (c * (tgroups // 2) + q) * N
        zeros16 = jnp.zeros((16,), jnp.float32)
        iota16 = lax.iota(jnp.int32, 16)
        gsem = (gs0, gs1)
        gdem = (gd0, gd1)
        iem = (ie0, ie1)
        ssem = (ss0, ss1)

        ebase = s * EP

        def adjust(par):
            for g in range(EB // 16):
                sl = pl.ds(g * 16, 16)
                eidxb[par, 0, sl] = eidxb[par, 0, sl] + coff

        def issue_gathers(par):
            pltpu.async_copy(st_hbm.at[eidxb.at[par, 0]], srcbuf.at[par],
                             gsem[par])
            pltpu.async_copy(dt_hbm.at[eidxb.at[par, 1]], dstbuf.at[par],
                             gdem[par])

        def wait_gathers(par):
            pltpu.make_async_copy(st_hbm.at[eidxb.at[par, 0]],
                                  srcbuf.at[par], gsem[par]).wait()
            pltpu.make_async_copy(dt_hbm.at[eidxb.at[par, 1]],
                                  dstbuf.at[par], gdem[par]).wait()

        def issue_idx(par, blk):
            pltpu.async_copy(
                eidx_hbm.at[pl.ds(0, 2), pl.ds(ebase + blk * EB, EB)],
                eidxb.at[par], iem[par])

        def wait_idx(par, blk):
            pltpu.make_async_copy(
                eidx_hbm.at[pl.ds(0, 2), pl.ds(ebase + blk * EB, EB)],
                eidxb.at[par], iem[par]).wait()

        def wait_scatter(par):
            pltpu.make_async_copy(msgbuf.at[par], acc.at[didxs.at[par]],
                                  ssem[par]).wait()

        hdr_mask = iota16 < KH
        align_idx = jnp.bitwise_and(iota16 + hd0, 15)
        nch = (R - 16) // 16
        chunk_heads = []
        for m in range(nch):
            pos = iota16 + (16 * m)
            hsel = jnp.zeros((16,), jnp.int32)
            for t in range(1, KH):
                hsel = hsel + jnp.where(pos >= t * C, 1, 0).astype(jnp.int32)
            chunk_heads.append(hsel)

        def compute(par):
            for j in range(EB):
                arow = srcbuf[par, j, pl.ds(0, 16)]
                drow = dstbuf[par, j, pl.ds(0, 16)]
                adst = drow.at[align_idx].get(mode="promise_in_bounds")
                e = arow + adst
                e = jnp.maximum(e, 0.2 * e)
                w = jnp.where(hdr_mask, jnp.exp(e), 0.0)
                msgbuf[par, j, pl.ds(0, 16)] = w
                for m in range(nch):
                    wv = w.at[chunk_heads[m]].get(mode="promise_in_bounds")
                    hv = srcbuf[par, j, pl.ds(16 + 16 * m, 16)]
                    msgbuf[par, j, pl.ds(16 + 16 * m, 16)] = hv * wv

        pltpu.sync_copy(eidx_hbm.at[pl.ds(0, 2), pl.ds(ebase, EB)],
                        eidxb.at[0])
        adjust(0)
        issue_gathers(0)
        issue_idx(1, 1)

        for j in range(ZR):
            for col0 in range(R // 16):
                zbuf[j, pl.ds(col0 * 16, 16)] = zeros16

        @pl.loop(0, RPS // ZR)
        def _zero(i):
            pltpu.sync_copy(zbuf, acc.at[pl.ds(s * RPS + i * ZR, ZR)])

        plsc.subcore_barrier()

        @pl.loop(0, NB // 2)
        def _pair(i):
            for P in range(2):
                b = 2 * i + P
                Q = 1 - P
                if P == 0:
                    wait_idx(Q, b + 1)
                    adjust(Q)
                    issue_gathers(Q)
                else:
                    @pl.when(i < NB // 2 - 1)
                    def _():
                        wait_idx(Q, b + 1)
                        adjust(Q)
                        issue_gathers(Q)

                wait_gathers(P)

                @pl.when(i > 0)
                def _():
                    wait_scatter(P)
                for g in range(EB // 16):
                    sl = pl.ds(g * 16, 16)
                    didxs[P, sl] = eidxb[P, 1, sl]

                @pl.when(i < NB // 2 - 1)
                def _():
                    issue_idx(P, b + 2)

                compute(P)
                pltpu.async_copy(msgbuf.at[P], acc.at[didxs.at[P]], ssem[P],
                                 add=True)

        wait_scatter(0)
        wait_scatter(1)
        plsc.subcore_barrier()

        row = s * RPS
        pltpu.sync_copy(acc.at[pl.ds(row, RPS)],
                        out_hbm.at[pl.ds(c * NP + row, RPS)])

    return sc_edge


def kernel(x, edge_index, W0, a_src0, a_dst0, b0, W1, a_src1, a_dst1, b1):
    eidx = edge_index.astype(jnp.int32)
    Asrc0 = _head_matrix(a_src0)
    Adst0 = _head_matrix(a_dst0)
    Asrc1 = _head_matrix(a_src1)
    Adst1 = _head_matrix(a_dst1)

    st0, dt0 = _prep0(x, W0, Asrc0, Adst0)
    acc0 = _make_sc_edge(80, 16, 4, 0, 2)(
        eidx, st0.reshape(2 * N, 80), dt0)
    st1, dt1 = _mid(acc0.reshape(2, NP, 80), b0.reshape(1, 128),
                    W1, Asrc1, Adst1)
    st1f = st1.reshape(4 * N, 96)
    accq0 = _make_sc_edge(96, 40, 2, 0, 4)(eidx, st1f, dt1)
    accq1 = _make_sc_edge(96, 40, 2, 1, 4)(eidx, st1f, dt1)
    return _final(accq0.reshape(2, NP, 96), accq1.reshape(2, NP, 96),
                  b1.reshape(1, 40))
